# baseline (speedup 1.0000x reference)
import jax
import jax.numpy as jnp
from jax import lax
from jax.experimental import pallas as pl
from jax.experimental.pallas import tpu as pltpu
from jax.experimental.pallas import tpu_sc as plsc

_IN_COLS = 256
_OUT_COLS = 512
_NC = 2
_NS = 16
_NW = _NC * _NS
_CHUNK = 32
_NIB = 4
_NOB = 2
_PREF = 3


_SC_ROWS = 131072


def _body(x_hbm, o_hbm, in_buf, out_buf, in_sem0, in_sem1, in_sem2,
          in_sem3, out_sem0, out_sem1):
    rows_per = _SC_ROWS // _NW
    n_chunks = rows_per // _CHUNK
    wid = lax.axis_index("s") * _NC + lax.axis_index("c")
    row_base = wid * rows_per

    in_sems = (in_sem0, in_sem1, in_sem2, in_sem3)
    out_sems = (out_sem0, out_sem1)

    lane = lax.broadcasted_iota(jnp.int32, (16,), 0)
    in_cols = [lane * 4 + 64 * g for g in range(4)]
    out_cols = [lane * 8 + 128 * g for g in range(4)]
    zeros16 = jnp.zeros((16,), jnp.float32)

    for b in range(_NOB):
        def zrow(r, carry, b=b):
            for g in range(_OUT_COLS // 16):
                out_buf[b, r, pl.ds(g * 16, 16)] = zeros16
            return carry
        lax.fori_loop(0, _CHUNK, zrow, 0)

    def in_copy(k, b):
        row0 = row_base + k * _CHUNK
        return pltpu.make_async_copy(
            x_hbm.at[pl.ds(row0, _CHUNK)], in_buf.at[b], in_sems[b])

    def out_copy(k, b):
        row0 = row_base + k * _CHUNK
        return pltpu.make_async_copy(
            out_buf.at[b], o_hbm.at[pl.ds(row0, _CHUNK)], out_sems[b])

    for j in range(_PREF):
        in_copy(j, j % _NIB).start()

    def step(cur, jin, jout):
        @pl.when(cur + _PREF < n_chunks)
        def _():
            in_copy(cur + _PREF, (jin + _PREF) % _NIB).start()
        in_copy(cur, jin).wait()

        @pl.when(cur >= _NOB)
        def _():
            out_copy(cur - _NOB, jout).wait()

        src = in_buf.at[jin]
        dst = out_buf.at[jout]

        def row_fn(r, carry):
            rs = jnp.full((16,), r, jnp.int32)
            for g in range(4):
                v = plsc.load_gather(src, [rs, in_cols[g]])
                plsc.store_scatter(dst, [rs, out_cols[g]], v)
            return carry
        lax.fori_loop(0, _CHUNK, row_fn, 0)

        out_copy(cur, jout).start()

    def outer(k, carry):
        for j in range(_NIB):
            step(k * _NIB + j, j, j % _NOB)
        return carry
    lax.fori_loop(0, n_chunks // _NIB, outer, 0)

    for b in range(_NOB):
        out_copy(n_chunks - _NOB + b, (n_chunks - _NOB + b) % _NOB).wait()


_TC_ROWS = 1024


def _tc_body(x_ref, o_ref):
    p = lax.broadcasted_iota(jnp.int32, (_IN_COLS, _OUT_COLS), 0)
    q = lax.broadcasted_iota(jnp.int32, (_IN_COLS, _OUT_COLS), 1)
    m = ((q % 8 == 0) & (2 * p == q)).astype(jnp.float32)
    o_ref[...] = jnp.dot(x_ref[...], m, preferred_element_type=jnp.float32)


def _tc_finish(input, sc_out):
    rows, _ = input.shape
    del sc_out
    base = _SC_ROWS // _TC_ROWS
    return pl.pallas_call(
        _tc_body,
        grid=((rows - _SC_ROWS) // _TC_ROWS,),
        in_specs=[
            pl.BlockSpec((_TC_ROWS, _IN_COLS), lambda i: (base + i, 0)),
        ],
        out_specs=pl.BlockSpec((_TC_ROWS, _OUT_COLS), lambda i: (i, 0)),
        out_shape=jax.ShapeDtypeStruct((rows - _SC_ROWS, _OUT_COLS),
                                       input.dtype),
    )(input)


def kernel(input):
    rows, cols = input.shape
    assert cols == _IN_COLS
    assert _SC_ROWS % (_NW * _NIB * _CHUNK) == 0
    assert (rows - _SC_ROWS) % _TC_ROWS == 0
    mesh = plsc.VectorSubcoreMesh(
        core_axis_name="c", subcore_axis_name="s", num_cores=_NC,
        num_subcores=_NS)
    f = pl.kernel(
        _body,
        out_type=jax.ShapeDtypeStruct((_SC_ROWS, _OUT_COLS), input.dtype),
        mesh=mesh,
        compiler_params=pltpu.CompilerParams(
            use_tc_tiling_on_sc=False, needs_layout_passes=False),
        scratch_types=[
            pltpu.VMEM((_NIB, _CHUNK, _IN_COLS), jnp.float32),
            pltpu.VMEM((_NOB, _CHUNK, _OUT_COLS), jnp.float32),
            pltpu.SemaphoreType.DMA,
            pltpu.SemaphoreType.DMA,
            pltpu.SemaphoreType.DMA,
            pltpu.SemaphoreType.DMA,
            pltpu.SemaphoreType.DMA,
            pltpu.SemaphoreType.DMA,
        ],
    )
    return jnp.concatenate([f(input), _tc_finish(input, None)], axis=0)


# speedup vs baseline: 1.0326x; 1.0326x over previous
import jax
import jax.numpy as jnp
from jax import lax
from jax.experimental import pallas as pl
from jax.experimental.pallas import tpu as pltpu
from jax.experimental.pallas import tpu_sc as plsc

_IN_COLS = 256
_OUT_COLS = 512
_NC = 2
_NS = 16
_NW = _NC * _NS
_CHUNK = 32
_NIB = 4
_NOB = 2
_PREF = 3


def _body(x_hbm, o_hbm, in_buf, out_buf, in_sem0, in_sem1, in_sem2,
          in_sem3, out_sem0, out_sem1):
    rows = o_hbm.shape[0]
    rows_per = rows // _NW
    n_chunks = rows_per // _CHUNK
    wid = lax.axis_index("s") * _NC + lax.axis_index("c")

    in_sems = (in_sem0, in_sem1, in_sem2, in_sem3)
    out_sems = (out_sem0, out_sem1)

    lane = lax.broadcasted_iota(jnp.int32, (16,), 0)
    in_cols = [lane * 4 + 64 * g for g in range(4)]
    out_cols = [lane * 8 + 128 * g for g in range(4)]
    zeros16 = jnp.zeros((16,), jnp.float32)

    for b in range(_NOB):
        def zrow(r, carry, b=b):
            for g in range(_OUT_COLS // 16):
                out_buf[b, r, pl.ds(g * 16, 16)] = zeros16
            return carry
        lax.fori_loop(0, _CHUNK, zrow, 0)

    def in_copy(k, b):
        row0 = (k * _NW + wid) * _CHUNK
        return pltpu.make_async_copy(
            x_hbm.at[pl.ds(row0, _CHUNK)], in_buf.at[b], in_sems[b])

    def out_copy(k, b):
        row0 = (k * _NW + wid) * _CHUNK
        return pltpu.make_async_copy(
            out_buf.at[b], o_hbm.at[pl.ds(row0, _CHUNK)], out_sems[b])

    for j in range(_PREF):
        in_copy(j, j % _NIB).start()

    def step(cur, jin, jout):
        @pl.when(cur + _PREF < n_chunks)
        def _():
            in_copy(cur + _PREF, (jin + _PREF) % _NIB).start()
        in_copy(cur, jin).wait()

        @pl.when(cur >= _NOB)
        def _():
            out_copy(cur - _NOB, jout).wait()

        src = in_buf.at[jin]
        dst = out_buf.at[jout]

        def row_fn(r, carry):
            rs = jnp.full((16,), r, jnp.int32)
            for g in range(4):
                v = plsc.load_gather(src, [rs, in_cols[g]])
                plsc.store_scatter(dst, [rs, out_cols[g]], v)
            return carry
        lax.fori_loop(0, _CHUNK, row_fn, 0)

        out_copy(cur, jout).start()

    def outer(k, carry):
        for j in range(_NIB):
            step(k * _NIB + j, j, j % _NOB)
        return carry
    lax.fori_loop(0, n_chunks // _NIB, outer, 0)

    for b in range(_NOB):
        out_copy(n_chunks - _NOB + b, (n_chunks - _NOB + b) % _NOB).wait()


def kernel(input):
    rows, cols = input.shape
    assert cols == _IN_COLS
    assert rows % (_NW * _NIB * _CHUNK) == 0
    mesh = plsc.VectorSubcoreMesh(
        core_axis_name="c", subcore_axis_name="s", num_cores=_NC,
        num_subcores=_NS)
    f = pl.kernel(
        _body,
        out_type=jax.ShapeDtypeStruct((rows, _OUT_COLS), input.dtype),
        mesh=mesh,
        compiler_params=pltpu.CompilerParams(
            use_tc_tiling_on_sc=False, needs_layout_passes=False),
        scratch_types=[
            pltpu.VMEM((_NIB, _CHUNK, _IN_COLS), jnp.float32),
            pltpu.VMEM((_NOB, _CHUNK, _OUT_COLS), jnp.float32),
            pltpu.SemaphoreType.DMA,
            pltpu.SemaphoreType.DMA,
            pltpu.SemaphoreType.DMA,
            pltpu.SemaphoreType.DMA,
            pltpu.SemaphoreType.DMA,
            pltpu.SemaphoreType.DMA,
        ],
    )
    return f(input)


# speedup vs baseline: 1.0434x; 1.0104x over previous
import jax
import jax.numpy as jnp
from jax import lax
from jax.experimental import pallas as pl
from jax.experimental.pallas import tpu as pltpu
from jax.experimental.pallas import tpu_sc as plsc

_IN_COLS = 256
_OUT_COLS = 512
_NC = 2
_NS = 16
_NW = _NC * _NS
_CHUNK = 64
_NIB = 2
_NOB = 2
_PREF = 1


def _body(x_hbm, o_hbm, in_buf, out_buf, in_sem0, in_sem1, in_sem2,
          in_sem3, out_sem0, out_sem1):
    rows = o_hbm.shape[0]
    rows_per = rows // _NW
    n_chunks = rows_per // _CHUNK
    wid = lax.axis_index("s") * _NC + lax.axis_index("c")
    row_base = wid * rows_per

    in_sems = (in_sem0, in_sem1, in_sem2, in_sem3)
    out_sems = (out_sem0, out_sem1)

    lane = lax.broadcasted_iota(jnp.int32, (16,), 0)
    in_cols = [lane * 4 + 64 * g for g in range(4)]
    out_cols = [lane * 8 + 128 * g for g in range(4)]
    zeros16 = jnp.zeros((16,), jnp.float32)

    for b in range(_NOB):
        def zrow(r, carry, b=b):
            for g in range(_OUT_COLS // 16):
                out_buf[b, r, pl.ds(g * 16, 16)] = zeros16
            return carry
        lax.fori_loop(0, _CHUNK, zrow, 0)

    def in_copy(k, b):
        row0 = row_base + k * _CHUNK
        return pltpu.make_async_copy(
            x_hbm.at[pl.ds(row0, _CHUNK)], in_buf.at[b], in_sems[b])

    def out_copy(k, b):
        row0 = row_base + k * _CHUNK
        return pltpu.make_async_copy(
            out_buf.at[b], o_hbm.at[pl.ds(row0, _CHUNK)], out_sems[b])

    for j in range(_PREF):
        in_copy(j, j % _NIB).start()

    def step(cur, jin, jout):
        @pl.when(cur + _PREF < n_chunks)
        def _():
            in_copy(cur + _PREF, (jin + _PREF) % _NIB).start()
        in_copy(cur, jin).wait()

        @pl.when(cur >= _NOB)
        def _():
            out_copy(cur - _NOB, jout).wait()

        src = in_buf.at[jin]
        dst = out_buf.at[jout]

        def row_fn(r, carry):
            rs = jnp.full((16,), r, jnp.int32)
            for g in range(4):
                v = plsc.load_gather(src, [rs, in_cols[g]])
                plsc.store_scatter(dst, [rs, out_cols[g]], v)
            return carry
        lax.fori_loop(0, _CHUNK, row_fn, 0)

        out_copy(cur, jout).start()

    def outer(k, carry):
        for j in range(_NIB):
            step(k * _NIB + j, j, j % _NOB)
        return carry
    lax.fori_loop(0, n_chunks // _NIB, outer, 0)

    for b in range(_NOB):
        out_copy(n_chunks - _NOB + b, (n_chunks - _NOB + b) % _NOB).wait()


def kernel(input):
    rows, cols = input.shape
    assert cols == _IN_COLS
    assert rows % (_NW * _NIB * _CHUNK) == 0
    mesh = plsc.VectorSubcoreMesh(
        core_axis_name="c", subcore_axis_name="s", num_cores=_NC,
        num_subcores=_NS)
    f = pl.kernel(
        _body,
        out_type=jax.ShapeDtypeStruct((rows, _OUT_COLS), input.dtype),
        mesh=mesh,
        compiler_params=pltpu.CompilerParams(
            use_tc_tiling_on_sc=False, needs_layout_passes=False),
        scratch_types=[
            pltpu.VMEM((_NIB, _CHUNK, _IN_COLS), jnp.float32),
            pltpu.VMEM((_NOB, _CHUNK, _OUT_COLS), jnp.float32),
            pltpu.SemaphoreType.DMA,
            pltpu.SemaphoreType.DMA,
            pltpu.SemaphoreType.DMA,
            pltpu.SemaphoreType.DMA,
            pltpu.SemaphoreType.DMA,
            pltpu.SemaphoreType.DMA,
        ],
    )
    return f(input)
